# Initial kernel scaffold; baseline (speedup 1.0000x reference)
import jax
import jax.numpy as jnp
from jax import lax
from jax.experimental import pallas as pl
from jax.experimental.pallas import tpu as pltpu
from jax.experimental.pallas import tpu_sc as plsc

N = 10000
D = 128
E = 80000
L1 = 64
L2 = 64

NPAD = 10240
EPAD = 81920
TRASH = N
NC, NS, LANES = 2, 16, 16
NTILES = NC * NS
EPT = EPAD // NTILES
CH = 128
NCHUNK = EPT // CH
RPT = NPAD // NS
ZW = 160

_mesh = plsc.VectorSubcoreMesh(core_axis_name="c", subcore_axis_name="s")


def _iota16():
    return lax.broadcasted_iota(jnp.int32, (LANES,), 0)


def _make_agg(with_cnt):
    out_type = [jax.ShapeDtypeStruct((NC, NPAD, D), jnp.float32)]
    scratch = [
        pltpu.VMEM_SHARED((NPAD, D), jnp.float32),
        pltpu.VMEM((CH,), jnp.int32),
        pltpu.VMEM((CH,), jnp.int32),
        pltpu.VMEM((CH,), jnp.int32),
        pltpu.VMEM((CH, D), jnp.float32),
    ]
    if with_cnt:
        out_type.append(jax.ShapeDtypeStruct((NC, NPAD, 16), jnp.float32))
        scratch += [
            pltpu.VMEM_SHARED((NPAD, 16), jnp.float32),
            pltpu.VMEM((CH, 16), jnp.float32),
        ]

    def body(*refs):
        if with_cnt:
            (table, rows_h, cols_h, zrow_h, zcnt_h, ones_h,
             out_s, out_c, acc, rbuf, cbuf, dbuf, gbuf, cacc, onesb) = refs
        else:
            (table, rows_h, cols_h, zrow_h,
             out_s, acc, rbuf, cbuf, dbuf, gbuf) = refs
        cid = lax.axis_index("c")
        sid = lax.axis_index("s")

        pltpu.sync_copy(zrow_h, acc.at[pl.ds(sid * RPT, RPT)])
        if with_cnt:
            pltpu.sync_copy(zcnt_h, cacc.at[pl.ds(sid * RPT, RPT)])
            pltpu.sync_copy(ones_h, onesb)
        plsc.subcore_barrier()

        def chunk(k, carry):
            base = (cid * NS + sid) * EPT + k * CH
            pltpu.sync_copy(rows_h.at[pl.ds(base, CH)], rbuf)
            pltpu.sync_copy(cols_h.at[pl.ds(base, CH)], cbuf)
            for g in range(CH // LANES):
                sl = pl.ds(g * LANES, LANES)
                r = rbuf[sl]
                c = cbuf[sl]
                dbuf[sl] = jnp.where(r == c, TRASH, r)
            pltpu.sync_copy(table.at[cbuf], gbuf)
            pltpu.sync_copy(gbuf, acc.at[dbuf], add=True)
            if with_cnt:
                pltpu.sync_copy(onesb, cacc.at[dbuf], add=True)
            return carry

        lax.fori_loop(0, NCHUNK, chunk, 0)
        plsc.subcore_barrier()

        sl = pl.ds(sid * RPT, RPT)
        pltpu.sync_copy(acc.at[sl], out_s.at[cid].at[sl])
        if with_cnt:
            pltpu.sync_copy(cacc.at[sl], out_c.at[cid].at[sl])

    return pl.kernel(body, out_type=tuple(out_type), mesh=_mesh,
                     scratch_types=scratch)


_agg_cnt = _make_agg(True)
_agg_sum = _make_agg(False)


RB = 256
GRID = NPAD // RB


def _l2t(u):
    n = jnp.sqrt(jnp.sum(u * u, axis=1, keepdims=True))
    return jnp.tanh(u / jnp.maximum(n, 1e-12))


def _base_tc_body(sp_ref, cp_ref, sn_ref, cn_ref, x_ref,
                  wp_ref, bp_ref, wn_ref, bn_ref, out_ref):
    x = x_ref[...]
    sp = sp_ref[0] + sp_ref[1]
    cp = jnp.maximum(cp_ref[0, :, 0:1] + cp_ref[1, :, 0:1], 1.0)
    aggp = sp / cp
    up = (jnp.dot(aggp, wp_ref[0:D, :], preferred_element_type=jnp.float32)
          + jnp.dot(x, wp_ref[D:2 * D, :], preferred_element_type=jnp.float32)
          + bp_ref[...])
    sn = sn_ref[0] + sn_ref[1]
    cn = jnp.maximum(cn_ref[0, :, 0:1] + cn_ref[1, :, 0:1], 1.0)
    aggn = sn / cn
    un = (jnp.dot(aggn, wn_ref[0:D, :], preferred_element_type=jnp.float32)
          + jnp.dot(x, wn_ref[D:2 * D, :], preferred_element_type=jnp.float32)
          + bn_ref[...])
    out_ref[...] = jnp.concatenate([_l2t(up), _l2t(un)], axis=1)


def _deep_tc_body(spd_ref, snd_ref, cp_ref, cn_ref, h_ref,
                  wp_ref, bp_ref, wn_ref, bn_ref, rw_ref, out_ref):
    h = h_ref[...]
    hp = h[:, :L1]
    hn = h[:, L1:]
    sp = spd_ref[0] + spd_ref[1]
    sn = snd_ref[0] + snd_ref[1]
    cp1 = cp_ref[0, :, 0:1] + cp_ref[1, :, 0:1] + 1.0
    cn1 = cn_ref[0, :, 0:1] + cn_ref[1, :, 0:1] + 1.0

    def head(o1, o2, xs, w_ref, b_ref):
        u = (jnp.dot(o1, w_ref[0:L1, :], preferred_element_type=jnp.float32)
             + jnp.dot(o2, w_ref[L1:2 * L1, :], preferred_element_type=jnp.float32)
             + jnp.dot(xs, w_ref[2 * L1:3 * L1, :], preferred_element_type=jnp.float32)
             + b_ref[...])
        return _l2t(u)

    zp = head((sp[:, :L1] + hp) / cp1, (sn[:, L1:] + hn) / cn1, hp, wp_ref, bp_ref)
    zn = head((sp[:, L1:] + hn) / cp1, (sn[:, :L1] + hp) / cn1, hn, wn_ref, bn_ref)
    z = jnp.concatenate([zp, zn], axis=1)
    u = jnp.dot(z, rw_ref[0:D, :], preferred_element_type=jnp.float32)
    v = jnp.dot(z, rw_ref[D:2 * D, :], preferred_element_type=jnp.float32)
    pad8 = jnp.zeros((RB, 8), jnp.float32)
    out_ref[...] = jnp.concatenate([z, u, pad8, v, pad8], axis=1)


def _row_spec(shape3):
    return pl.BlockSpec((NC, RB) + shape3, lambda i: (0, i, 0))


def _full_spec(shape):
    nd = len(shape)
    return pl.BlockSpec(shape, lambda i, _n=nd: (0,) * _n)


_base_tc = pl.pallas_call(
    _base_tc_body,
    grid=(GRID,),
    in_specs=[
        _row_spec((D,)), _row_spec((16,)), _row_spec((D,)), _row_spec((16,)),
        pl.BlockSpec((RB, D), lambda i: (i, 0)),
        _full_spec((2 * D, L1)), _full_spec((1, L1)),
        _full_spec((2 * D, L1)), _full_spec((1, L1)),
    ],
    out_specs=pl.BlockSpec((RB, 2 * L1), lambda i: (i, 0)),
    out_shape=jax.ShapeDtypeStruct((NPAD, 2 * L1), jnp.float32),
)

_deep_tc = pl.pallas_call(
    _deep_tc_body,
    grid=(GRID,),
    in_specs=[
        _row_spec((D,)), _row_spec((D,)), _row_spec((16,)), _row_spec((16,)),
        pl.BlockSpec((RB, 2 * L1), lambda i: (i, 0)),
        _full_spec((3 * L1, L2)), _full_spec((1, L2)),
        _full_spec((3 * L1, L2)), _full_spec((1, L2)),
        _full_spec((2 * L2, 8)),
    ],
    out_specs=pl.BlockSpec((RB, ZW), lambda i: (i, 0)),
    out_shape=jax.ShapeDtypeStruct((NPAD, ZW), jnp.float32),
)


_LN2 = 0.6931471805599453


def _fastlog(s):
    bits = lax.bitcast_convert_type(s, jnp.int32)
    e = ((bits >> 23) & 0xFF).astype(jnp.float32) - 127.0
    m = lax.bitcast_convert_type((bits & 0x7FFFFF) | 0x3F800000, jnp.float32)
    t = (m - 1.0) / (m + 1.0)
    t2 = t * t
    poly = 1.0 + t2 * (1.0 / 3.0 + t2 * (1.0 / 5.0 + t2 * (1.0 / 7.0 + t2 / 9.0)))
    return e * _LN2 + 2.0 * t * poly


def _loss_body(zext_h, pi_h, pj_h, pk_h, ni_h, nj_h, nk_h, t6_h,
               out_reg, out_relu,
               ibuf, jbuf, kbuf, bi, bj, bk, tb0, tb1, tb2, accb):
    cid = lax.axis_index("c")
    sid = lax.axis_index("s")
    wid = cid * NS + sid
    iota = _iota16()
    z16 = jnp.zeros((LANES,), jnp.float32)
    tbs = (tb0, tb1, tb2)

    acc_reg = z16
    acc_relu = z16
    fams = [
        (pi_h, pj_h, pk_h, (0, 4, 5), 1.0),
        (ni_h, nj_h, nk_h, (1, 2, 3), -1.0),
    ]
    for ih, jh, kh, blocks, sign in fams:
        def chunk(k, carry, ih=ih, jh=jh, kh=kh, blocks=blocks, sign=sign):
            acc_reg, acc_relu = carry
            base = wid * EPT + k * CH
            pltpu.sync_copy(ih.at[pl.ds(base, CH)], ibuf)
            pltpu.sync_copy(jh.at[pl.ds(base, CH)], jbuf)
            pltpu.sync_copy(kh.at[pl.ds(base, CH)], kbuf)
            pltpu.sync_copy(zext_h.at[ibuf], bi)
            pltpu.sync_copy(zext_h.at[jbuf], bj)
            pltpu.sync_copy(zext_h.at[kbuf], bk)
            for o, blk in enumerate(blocks):
                pltpu.sync_copy(t6_h.at[blk].at[pl.ds(base, CH)], tbs[o])
            for g in range(CH // LANES):
                rows = iota + g * LANES

                def dstep(jj, c2):
                    aij, aik = c2
                    for dd in range(8):
                        col = jnp.zeros((LANES,), jnp.int32) + (jj * 8 + dd)
                        a = plsc.load_gather(bi, [rows, col])
                        b = plsc.load_gather(bj, [rows, col])
                        c = plsc.load_gather(bk, [rows, col])
                        aij = aij + (a - b) * (a - b)
                        aik = aik + (a - c) * (a - c)
                    return aij, aik

                aij, aik = lax.fori_loop(0, D // 8, dstep, (z16, z16))
                acc_relu = acc_relu + jnp.maximum(sign * (aij - aik), 0.0)

                validf = jnp.where(base + rows < E, 1.0, 0.0)
                for o, (ba, bb) in enumerate(((bi, bj), (bi, bk), (bj, bk))):
                    p = []
                    for ci in range(3):
                        cu = jnp.zeros((LANES,), jnp.int32) + (D + ci)
                        cv = jnp.zeros((LANES,), jnp.int32) + (D + 16 + ci)
                        p.append(plsc.load_gather(ba, [rows, cu])
                                 + plsc.load_gather(bb, [rows, cv]))
                    m = jnp.maximum(p[0], jnp.maximum(p[1], p[2]))
                    s = (jnp.exp(p[0] - m) + jnp.exp(p[1] - m)
                         + jnp.exp(p[2] - m))
                    ls = m + _fastlog(s)
                    t = tbs[o][pl.ds(g * LANES, LANES)]
                    pt = jnp.where(t == 0, p[0], jnp.where(t == 1, p[1], p[2]))
                    acc_reg = acc_reg + (ls - pt) * validf
            return acc_reg, acc_relu

        acc_reg, acc_relu = lax.fori_loop(0, NCHUNK, chunk,
                                          (acc_reg, acc_relu))

    accb[0, :] = acc_reg
    pltpu.sync_copy(accb, out_reg.at[pl.ds(wid, 1)])
    accb[0, :] = acc_relu
    pltpu.sync_copy(accb, out_relu.at[pl.ds(wid, 1)])


_loss_sc = pl.kernel(
    _loss_body,
    out_type=(jax.ShapeDtypeStruct((NTILES, 16), jnp.float32),
              jax.ShapeDtypeStruct((NTILES, 16), jnp.float32)),
    mesh=_mesh,
    scratch_types=[
        pltpu.VMEM((CH,), jnp.int32),
        pltpu.VMEM((CH,), jnp.int32),
        pltpu.VMEM((CH,), jnp.int32),
        pltpu.VMEM((CH, ZW), jnp.float32),
        pltpu.VMEM((CH, ZW), jnp.float32),
        pltpu.VMEM((CH, ZW), jnp.float32),
        pltpu.VMEM((CH,), jnp.int32),
        pltpu.VMEM((CH,), jnp.int32),
        pltpu.VMEM((CH,), jnp.int32),
        pltpu.VMEM((1, 16), jnp.float32),
    ],
)


def kernel(X, positive_edges, negative_edges, target, pos_surrogates,
           neg_surrogates, W_pos_base, b_pos_base, W_neg_base, b_neg_base,
           W_pos_deep, b_pos_deep, W_neg_deep, b_neg_deep,
           regression_weights):
    f32 = jnp.float32
    padE = lambda a: jnp.pad(a, (0, EPAD - E))
    rp, cp_ = padE(positive_edges[0]), padE(positive_edges[1])
    rn, cn_ = padE(negative_edges[0]), padE(negative_edges[1])
    pk = padE(pos_surrogates)
    nk = padE(neg_surrogates)
    t6 = jnp.pad(target.reshape(6, E), ((0, 0), (0, EPAD - E)))
    Xp = jnp.pad(X, ((0, NPAD - N), (0, 0)))

    zrow = jnp.zeros((RPT, D), f32)
    zcnt = jnp.zeros((RPT, 16), f32)
    ones_pat = jnp.zeros((CH, 16), f32).at[:, 0].set(1.0)

    sp, cpc = _agg_cnt(Xp, rp, cp_, zrow, zcnt, ones_pat)
    sn, cnc = _agg_cnt(Xp, rn, cn_, zrow, zcnt, ones_pat)
    H = _base_tc(sp, cpc, sn, cnc, Xp,
                 W_pos_base, b_pos_base.reshape(1, L1),
                 W_neg_base, b_neg_base.reshape(1, L1))
    (spd,) = _agg_sum(H, rp, cp_, zrow)
    (snd,) = _agg_sum(H, rn, cn_, zrow)
    rw8 = jnp.pad(regression_weights, ((0, 0), (0, 5)))
    Zext = _deep_tc(spd, snd, cpc, cnc, H,
                    W_pos_deep, b_pos_deep.reshape(1, L2),
                    W_neg_deep, b_neg_deep.reshape(1, L2), rw8)
    reg, relu = _loss_sc(Zext, rp, cp_, pk, rn, cn_, nk, t6)
    z = Zext[:N, :D]
    loss = jnp.sum(reg) / (6.0 * E) + jnp.sum(relu) / E
    return loss, z


# speedup vs baseline: 1.4355x; 1.4355x over previous
import jax
import jax.numpy as jnp
from jax import lax
from jax.experimental import pallas as pl
from jax.experimental.pallas import tpu as pltpu
from jax.experimental.pallas import tpu_sc as plsc

N = 10000
D = 128
E = 80000
L1 = 64
L2 = 64

NPAD = 10240
EPAD = 81920
TRASH = N
NC, NS, LANES = 2, 16, 16
NTILES = NC * NS
EPT = EPAD // NTILES
CH = 128
NCHUNK = EPT // CH
RPT = NPAD // NS
ZW = 160

_mesh = plsc.VectorSubcoreMesh(core_axis_name="c", subcore_axis_name="s")


def _iota16():
    return lax.broadcasted_iota(jnp.int32, (LANES,), 0)


def _make_agg(with_cnt):
    out_type = [jax.ShapeDtypeStruct((NC, NPAD, D), jnp.float32)]
    scratch = [
        pltpu.VMEM_SHARED((NPAD, D), jnp.float32),
        pltpu.VMEM((CH,), jnp.int32),
        pltpu.VMEM((CH,), jnp.int32),
        pltpu.VMEM((CH,), jnp.int32),
        pltpu.VMEM((CH, D), jnp.float32),
    ]
    if with_cnt:
        out_type.append(jax.ShapeDtypeStruct((NTILES, NPAD), jnp.float32))
        scratch += [
            pltpu.VMEM((NPAD,), jnp.float32),
        ]

    def body(*refs):
        if with_cnt:
            (table, rows_h, cols_h, zrow_h, zcnt_h,
             out_s, out_c, acc, rbuf, cbuf, dbuf, gbuf, hist) = refs
        else:
            (table, rows_h, cols_h, zrow_h,
             out_s, acc, rbuf, cbuf, dbuf, gbuf) = refs
        cid = lax.axis_index("c")
        sid = lax.axis_index("s")
        wid = cid * NS + sid

        pltpu.sync_copy(zrow_h, acc.at[pl.ds(sid * RPT, RPT)])
        if with_cnt:
            pltpu.sync_copy(zcnt_h, hist)
        plsc.subcore_barrier()
        ones16 = jnp.ones((LANES,), jnp.float32)

        def chunk(k, carry):
            base = wid * EPT + k * CH
            pltpu.sync_copy(rows_h.at[pl.ds(base, CH)], rbuf)
            pltpu.sync_copy(cols_h.at[pl.ds(base, CH)], cbuf)
            for g in range(CH // LANES):
                sl = pl.ds(g * LANES, LANES)
                r = rbuf[sl]
                c = cbuf[sl]
                d = jnp.where(r == c, TRASH, r)
                dbuf[sl] = d
                if with_cnt:
                    plsc.addupdate_scatter(hist, [d], ones16)
            pltpu.sync_copy(table.at[cbuf], gbuf)
            pltpu.sync_copy(gbuf, acc.at[dbuf], add=True)
            return carry

        lax.fori_loop(0, NCHUNK, chunk, 0)
        plsc.subcore_barrier()

        sl = pl.ds(sid * RPT, RPT)
        pltpu.sync_copy(acc.at[sl], out_s.at[cid].at[sl])
        if with_cnt:
            pltpu.sync_copy(hist, out_c.at[wid])

    return pl.kernel(body, out_type=tuple(out_type), mesh=_mesh,
                     compiler_params=pltpu.CompilerParams(
                         needs_layout_passes=False),
                     scratch_types=scratch)


_agg_cnt = _make_agg(True)
_agg_sum = _make_agg(False)


RB = 256
GRID = NPAD // RB


def _l2t(u):
    n = jnp.sqrt(jnp.sum(u * u, axis=1, keepdims=True))
    return jnp.tanh(u / jnp.maximum(n, 1e-12))


def _base_tc_body(sp_ref, cp_ref, sn_ref, cn_ref, x_ref,
                  wp_ref, bp_ref, wn_ref, bn_ref, out_ref):
    x = x_ref[...]
    sp = sp_ref[0] + sp_ref[1]
    cp = jnp.maximum(cp_ref[...], 1.0)
    aggp = sp / cp
    up = (jnp.dot(aggp, wp_ref[0:D, :], preferred_element_type=jnp.float32)
          + jnp.dot(x, wp_ref[D:2 * D, :], preferred_element_type=jnp.float32)
          + bp_ref[...])
    sn = sn_ref[0] + sn_ref[1]
    cn = jnp.maximum(cn_ref[...], 1.0)
    aggn = sn / cn
    un = (jnp.dot(aggn, wn_ref[0:D, :], preferred_element_type=jnp.float32)
          + jnp.dot(x, wn_ref[D:2 * D, :], preferred_element_type=jnp.float32)
          + bn_ref[...])
    out_ref[...] = jnp.concatenate([_l2t(up), _l2t(un)], axis=1)


def _deep_tc_body(spd_ref, snd_ref, cp_ref, cn_ref, h_ref,
                  wp_ref, bp_ref, wn_ref, bn_ref, rw_ref, out_ref, uv_ref):
    h = h_ref[...]
    hp = h[:, :L1]
    hn = h[:, L1:]
    sp = spd_ref[0] + spd_ref[1]
    sn = snd_ref[0] + snd_ref[1]
    cp1 = cp_ref[...] + 1.0
    cn1 = cn_ref[...] + 1.0

    def head(o1, o2, xs, w_ref, b_ref):
        u = (jnp.dot(o1, w_ref[0:L1, :], preferred_element_type=jnp.float32)
             + jnp.dot(o2, w_ref[L1:2 * L1, :], preferred_element_type=jnp.float32)
             + jnp.dot(xs, w_ref[2 * L1:3 * L1, :], preferred_element_type=jnp.float32)
             + b_ref[...])
        return _l2t(u)

    zp = head((sp[:, :L1] + hp) / cp1, (sn[:, L1:] + hn) / cn1, hp, wp_ref, bp_ref)
    zn = head((sp[:, L1:] + hn) / cp1, (sn[:, :L1] + hp) / cn1, hn, wn_ref, bn_ref)
    z = jnp.concatenate([zp, zn], axis=1)
    out_ref[...] = z
    uv_ref[...] = jnp.dot(z, rw_ref[...], preferred_element_type=jnp.float32)


def _row_spec(shape3):
    return pl.BlockSpec((NC, RB) + shape3, lambda i: (0, i, 0))


def _col_spec():
    return pl.BlockSpec((RB, 1), lambda i: (i, 0))


def _full_spec(shape):
    nd = len(shape)
    return pl.BlockSpec(shape, lambda i, _n=nd: (0,) * _n)


_base_tc = pl.pallas_call(
    _base_tc_body,
    grid=(GRID,),
    in_specs=[
        _row_spec((D,)), _col_spec(), _row_spec((D,)), _col_spec(),
        pl.BlockSpec((RB, D), lambda i: (i, 0)),
        _full_spec((2 * D, L1)), _full_spec((1, L1)),
        _full_spec((2 * D, L1)), _full_spec((1, L1)),
    ],
    out_specs=pl.BlockSpec((RB, 2 * L1), lambda i: (i, 0)),
    out_shape=jax.ShapeDtypeStruct((NPAD, 2 * L1), jnp.float32),
)

_deep_tc = pl.pallas_call(
    _deep_tc_body,
    grid=(GRID,),
    in_specs=[
        _row_spec((D,)), _row_spec((D,)), _col_spec(), _col_spec(),
        pl.BlockSpec((RB, 2 * L1), lambda i: (i, 0)),
        _full_spec((3 * L1, L2)), _full_spec((1, L2)),
        _full_spec((3 * L1, L2)), _full_spec((1, L2)),
        _full_spec((D, 8)),
    ],
    out_specs=(pl.BlockSpec((RB, D), lambda i: (i, 0)),
               pl.BlockSpec((RB, 8), lambda i: (i, 0))),
    out_shape=(jax.ShapeDtypeStruct((NPAD, D), jnp.float32),
               jax.ShapeDtypeStruct((NPAD, 8), jnp.float32)),
)


_LN2 = 0.6931471805599453


def _fastlog(s):
    bits = lax.bitcast_convert_type(s, jnp.int32)
    e = ((bits >> 23) & 0xFF).astype(jnp.float32) - 127.0
    m = lax.bitcast_convert_type((bits & 0x7FFFFF) | 0x3F800000, jnp.float32)
    t = (m - 1.0) / (m + 1.0)
    t2 = t * t
    poly = 1.0 + t2 * (1.0 / 3.0 + t2 * (1.0 / 5.0 + t2 * (1.0 / 7.0 + t2 / 9.0)))
    return e * _LN2 + 2.0 * t * poly


def _loss_body(z_h, uv6_h, pi_h, pj_h, pk_h, ni_h, nj_h, nk_h, t6_h,
               out_reg, out_relu,
               ibuf, jbuf, kbuf, bi, bj, bk, u0, u1, u2, v0, v1, v2,
               tb0, tb1, tb2, accb):
    cid = lax.axis_index("c")
    sid = lax.axis_index("s")
    wid = cid * NS + sid
    iota = _iota16()
    z16 = jnp.zeros((LANES,), jnp.float32)
    tbs = (tb0, tb1, tb2)

    uvt = (u0, u1, u2, v0, v1, v2)
    for c in range(6):
        pltpu.sync_copy(uv6_h.at[c], uvt[c])

    acc_reg = z16
    acc_relu = z16
    fams = [
        (pi_h, pj_h, pk_h, (0, 4, 5), 1.0),
        (ni_h, nj_h, nk_h, (1, 2, 3), -1.0),
    ]
    for ih, jh, kh, blocks, sign in fams:
        def chunk(k, carry, ih=ih, jh=jh, kh=kh, blocks=blocks, sign=sign):
            acc_reg, acc_relu = carry
            base = wid * EPT + k * CH
            pltpu.sync_copy(ih.at[pl.ds(base, CH)], ibuf)
            pltpu.sync_copy(jh.at[pl.ds(base, CH)], jbuf)
            pltpu.sync_copy(kh.at[pl.ds(base, CH)], kbuf)
            pltpu.sync_copy(z_h.at[ibuf], bi)
            pltpu.sync_copy(z_h.at[jbuf], bj)
            pltpu.sync_copy(z_h.at[kbuf], bk)
            for o, blk in enumerate(blocks):
                pltpu.sync_copy(t6_h.at[blk].at[pl.ds(base, CH)], tbs[o])
            for g in range(CH // LANES):
                rows = iota + g * LANES

                def dstep(jj, c2):
                    aij, aik = c2
                    for dd in range(8):
                        col = jnp.zeros((LANES,), jnp.int32) + (jj * 8 + dd)
                        a = plsc.load_gather(bi, [rows, col])
                        b = plsc.load_gather(bj, [rows, col])
                        c = plsc.load_gather(bk, [rows, col])
                        aij = aij + (a - b) * (a - b)
                        aik = aik + (a - c) * (a - c)
                    return aij, aik

                aij, aik = lax.fori_loop(0, D // 8, dstep, (z16, z16))
                acc_relu = acc_relu + jnp.maximum(sign * (aij - aik), 0.0)

                validf = jnp.where(base + rows < E, 1.0, 0.0)
                sl = pl.ds(g * LANES, LANES)
                na, nb, nk_ = ibuf[sl], jbuf[sl], kbuf[sl]
                for o, (xa, xb) in enumerate(((na, nb), (na, nk_), (nb, nk_))):
                    p = [plsc.load_gather(uvt[ci], [xa])
                         + plsc.load_gather(uvt[3 + ci], [xb])
                         for ci in range(3)]
                    m = jnp.maximum(p[0], jnp.maximum(p[1], p[2]))
                    s = (jnp.exp(p[0] - m) + jnp.exp(p[1] - m)
                         + jnp.exp(p[2] - m))
                    ls = m + _fastlog(s)
                    t = tbs[o][pl.ds(g * LANES, LANES)]
                    pt = jnp.where(t == 0, p[0], jnp.where(t == 1, p[1], p[2]))
                    acc_reg = acc_reg + (ls - pt) * validf
            return acc_reg, acc_relu

        acc_reg, acc_relu = lax.fori_loop(0, NCHUNK, chunk,
                                          (acc_reg, acc_relu))

    accb[0, :] = acc_reg
    pltpu.sync_copy(accb, out_reg.at[pl.ds(wid, 1)])
    accb[0, :] = acc_relu
    pltpu.sync_copy(accb, out_relu.at[pl.ds(wid, 1)])


_loss_sc = pl.kernel(
    _loss_body,
    out_type=(jax.ShapeDtypeStruct((NTILES, 16), jnp.float32),
              jax.ShapeDtypeStruct((NTILES, 16), jnp.float32)),
    mesh=_mesh,
    compiler_params=pltpu.CompilerParams(needs_layout_passes=False),
    scratch_types=[
        pltpu.VMEM((CH,), jnp.int32),
        pltpu.VMEM((CH,), jnp.int32),
        pltpu.VMEM((CH,), jnp.int32),
        pltpu.VMEM((CH, D), jnp.float32),
        pltpu.VMEM((CH, D), jnp.float32),
        pltpu.VMEM((CH, D), jnp.float32),
        pltpu.VMEM((NPAD,), jnp.float32),
        pltpu.VMEM((NPAD,), jnp.float32),
        pltpu.VMEM((NPAD,), jnp.float32),
        pltpu.VMEM((NPAD,), jnp.float32),
        pltpu.VMEM((NPAD,), jnp.float32),
        pltpu.VMEM((NPAD,), jnp.float32),
        pltpu.VMEM((CH,), jnp.int32),
        pltpu.VMEM((CH,), jnp.int32),
        pltpu.VMEM((CH,), jnp.int32),
        pltpu.VMEM((1, 16), jnp.float32),
    ],
)


def kernel(X, positive_edges, negative_edges, target, pos_surrogates,
           neg_surrogates, W_pos_base, b_pos_base, W_neg_base, b_neg_base,
           W_pos_deep, b_pos_deep, W_neg_deep, b_neg_deep,
           regression_weights):
    f32 = jnp.float32
    padE = lambda a: jnp.pad(a, (0, EPAD - E))
    rp, cp_ = padE(positive_edges[0]), padE(positive_edges[1])
    rn, cn_ = padE(negative_edges[0]), padE(negative_edges[1])
    pk = padE(pos_surrogates)
    nk = padE(neg_surrogates)
    t6 = jnp.pad(target.reshape(6, E), ((0, 0), (0, EPAD - E)))
    Xp = jnp.pad(X, ((0, NPAD - N), (0, 0)))

    zrow = jnp.zeros((RPT, D), f32)
    zcnt = jnp.zeros((NPAD,), f32)

    sp, cp32 = _agg_cnt(Xp, rp, cp_, zrow, zcnt)
    sn, cn32 = _agg_cnt(Xp, rn, cn_, zrow, zcnt)
    cpc = jnp.sum(cp32, axis=0)[:, None]
    cnc = jnp.sum(cn32, axis=0)[:, None]
    H = _base_tc(sp, cpc, sn, cnc, Xp,
                 W_pos_base, b_pos_base.reshape(1, L1),
                 W_neg_base, b_neg_base.reshape(1, L1))
    (spd,) = _agg_sum(H, rp, cp_, zrow)
    (snd,) = _agg_sum(H, rn, cn_, zrow)
    rw8 = jnp.pad(jnp.concatenate(
        [regression_weights[:D], regression_weights[D:]], axis=1),
        ((0, 0), (0, 2)))
    Z, UVo = _deep_tc(spd, snd, cpc, cnc, H,
                      W_pos_deep, b_pos_deep.reshape(1, L2),
                      W_neg_deep, b_neg_deep.reshape(1, L2), rw8)
    uv6 = UVo[:, :6].T
    reg, relu = _loss_sc(Z, uv6, rp, cp_, pk, rn, cn_, nk, t6)
    z = Z[:N]
    loss = jnp.sum(reg) / (6.0 * E) + jnp.sum(relu) / E
    return loss, z


# speedup vs baseline: 1.4366x; 1.0008x over previous
import jax
import jax.numpy as jnp
from jax import lax
from jax.experimental import pallas as pl
from jax.experimental.pallas import tpu as pltpu
from jax.experimental.pallas import tpu_sc as plsc

N = 10000
D = 128
E = 80000
L1 = 64
L2 = 64

NPAD = 10240
EPAD = 81920
TRASH = N
NC, NS, LANES = 2, 16, 16
NTILES = NC * NS
EPT = EPAD // NTILES
CH = 128
NCHUNK = EPT // CH
RPT = NPAD // NS
ZW = 160

_mesh = plsc.VectorSubcoreMesh(core_axis_name="c", subcore_axis_name="s")


def _iota16():
    return lax.broadcasted_iota(jnp.int32, (LANES,), 0)


def _make_agg(with_cnt):
    out_type = [jax.ShapeDtypeStruct((NC, NPAD, D), jnp.float32)]
    scratch = [
        pltpu.VMEM_SHARED((NPAD, D), jnp.float32),
        pltpu.VMEM((CH,), jnp.int32),
        pltpu.VMEM((CH,), jnp.int32),
        pltpu.VMEM((CH,), jnp.int32),
        pltpu.VMEM((CH, D), jnp.float32),
    ]
    if with_cnt:
        out_type.append(jax.ShapeDtypeStruct((NTILES, NPAD), jnp.float32))
        scratch += [
            pltpu.VMEM((NPAD,), jnp.float32),
        ]

    def body(*refs):
        if with_cnt:
            (table, rows_h, cols_h, zrow_h, zcnt_h,
             out_s, out_c, acc, rbuf, cbuf, dbuf, gbuf, hist) = refs
        else:
            (table, rows_h, cols_h, zrow_h,
             out_s, acc, rbuf, cbuf, dbuf, gbuf) = refs
        cid = lax.axis_index("c")
        sid = lax.axis_index("s")
        wid = cid * NS + sid

        pltpu.sync_copy(zrow_h, acc.at[pl.ds(sid * RPT, RPT)])
        if with_cnt:
            pltpu.sync_copy(zcnt_h, hist)
        plsc.subcore_barrier()
        ones16 = jnp.ones((LANES,), jnp.float32)

        def chunk(k, carry):
            base = wid * EPT + k * CH
            pltpu.sync_copy(rows_h.at[pl.ds(base, CH)], rbuf)
            pltpu.sync_copy(cols_h.at[pl.ds(base, CH)], cbuf)
            for g in range(CH // LANES):
                sl = pl.ds(g * LANES, LANES)
                r = rbuf[sl]
                c = cbuf[sl]
                d = jnp.where(r == c, TRASH, r)
                dbuf[sl] = d
                if with_cnt:
                    plsc.addupdate_scatter(hist, [d], ones16)
            pltpu.sync_copy(table.at[cbuf], gbuf)
            pltpu.sync_copy(gbuf, acc.at[dbuf], add=True)
            return carry

        lax.fori_loop(0, NCHUNK, chunk, 0)
        plsc.subcore_barrier()

        sl = pl.ds(sid * RPT, RPT)
        pltpu.sync_copy(acc.at[sl], out_s.at[cid].at[sl])
        if with_cnt:
            pltpu.sync_copy(hist, out_c.at[wid])

    return pl.kernel(body, out_type=tuple(out_type), mesh=_mesh,
                     compiler_params=pltpu.CompilerParams(
                         needs_layout_passes=False,
                         disable_bounds_checks=True),
                     scratch_types=scratch)


_agg_cnt = _make_agg(True)
_agg_sum = _make_agg(False)


RB = 256
GRID = NPAD // RB


def _l2t(u):
    n = jnp.sqrt(jnp.sum(u * u, axis=1, keepdims=True))
    return jnp.tanh(u / jnp.maximum(n, 1e-12))


def _base_tc_body(sp_ref, cp_ref, sn_ref, cn_ref, x_ref,
                  wp_ref, bp_ref, wn_ref, bn_ref, out_ref):
    x = x_ref[...]
    sp = sp_ref[0] + sp_ref[1]
    cp = jnp.maximum(cp_ref[...], 1.0)
    aggp = sp / cp
    up = (jnp.dot(aggp, wp_ref[0:D, :], preferred_element_type=jnp.float32)
          + jnp.dot(x, wp_ref[D:2 * D, :], preferred_element_type=jnp.float32)
          + bp_ref[...])
    sn = sn_ref[0] + sn_ref[1]
    cn = jnp.maximum(cn_ref[...], 1.0)
    aggn = sn / cn
    un = (jnp.dot(aggn, wn_ref[0:D, :], preferred_element_type=jnp.float32)
          + jnp.dot(x, wn_ref[D:2 * D, :], preferred_element_type=jnp.float32)
          + bn_ref[...])
    out_ref[...] = jnp.concatenate([_l2t(up), _l2t(un)], axis=1)


def _deep_tc_body(spd_ref, snd_ref, cp_ref, cn_ref, h_ref,
                  wp_ref, bp_ref, wn_ref, bn_ref, rw_ref, out_ref, uv_ref):
    h = h_ref[...]
    hp = h[:, :L1]
    hn = h[:, L1:]
    sp = spd_ref[0] + spd_ref[1]
    sn = snd_ref[0] + snd_ref[1]
    cp1 = cp_ref[...] + 1.0
    cn1 = cn_ref[...] + 1.0

    def head(o1, o2, xs, w_ref, b_ref):
        u = (jnp.dot(o1, w_ref[0:L1, :], preferred_element_type=jnp.float32)
             + jnp.dot(o2, w_ref[L1:2 * L1, :], preferred_element_type=jnp.float32)
             + jnp.dot(xs, w_ref[2 * L1:3 * L1, :], preferred_element_type=jnp.float32)
             + b_ref[...])
        return _l2t(u)

    zp = head((sp[:, :L1] + hp) / cp1, (sn[:, L1:] + hn) / cn1, hp, wp_ref, bp_ref)
    zn = head((sp[:, L1:] + hn) / cp1, (sn[:, :L1] + hp) / cn1, hn, wn_ref, bn_ref)
    z = jnp.concatenate([zp, zn], axis=1)
    out_ref[...] = z
    uv_ref[...] = jnp.dot(z, rw_ref[...], preferred_element_type=jnp.float32)


def _row_spec(shape3):
    return pl.BlockSpec((NC, RB) + shape3, lambda i: (0, i, 0))


def _col_spec():
    return pl.BlockSpec((RB, 1), lambda i: (i, 0))


def _full_spec(shape):
    nd = len(shape)
    return pl.BlockSpec(shape, lambda i, _n=nd: (0,) * _n)


_base_tc = pl.pallas_call(
    _base_tc_body,
    grid=(GRID,),
    in_specs=[
        _row_spec((D,)), _col_spec(), _row_spec((D,)), _col_spec(),
        pl.BlockSpec((RB, D), lambda i: (i, 0)),
        _full_spec((2 * D, L1)), _full_spec((1, L1)),
        _full_spec((2 * D, L1)), _full_spec((1, L1)),
    ],
    out_specs=pl.BlockSpec((RB, 2 * L1), lambda i: (i, 0)),
    out_shape=jax.ShapeDtypeStruct((NPAD, 2 * L1), jnp.float32),
)

_deep_tc = pl.pallas_call(
    _deep_tc_body,
    grid=(GRID,),
    in_specs=[
        _row_spec((D,)), _row_spec((D,)), _col_spec(), _col_spec(),
        pl.BlockSpec((RB, 2 * L1), lambda i: (i, 0)),
        _full_spec((3 * L1, L2)), _full_spec((1, L2)),
        _full_spec((3 * L1, L2)), _full_spec((1, L2)),
        _full_spec((D, 8)),
    ],
    out_specs=(pl.BlockSpec((RB, D), lambda i: (i, 0)),
               pl.BlockSpec((RB, 8), lambda i: (i, 0))),
    out_shape=(jax.ShapeDtypeStruct((NPAD, D), jnp.float32),
               jax.ShapeDtypeStruct((NPAD, 8), jnp.float32)),
)


_LN2 = 0.6931471805599453


def _fastlog(s):
    bits = lax.bitcast_convert_type(s, jnp.int32)
    e = ((bits >> 23) & 0xFF).astype(jnp.float32) - 127.0
    m = lax.bitcast_convert_type((bits & 0x7FFFFF) | 0x3F800000, jnp.float32)
    t = (m - 1.0) / (m + 1.0)
    t2 = t * t
    poly = 1.0 + t2 * (1.0 / 3.0 + t2 * (1.0 / 5.0 + t2 * (1.0 / 7.0 + t2 / 9.0)))
    return e * _LN2 + 2.0 * t * poly


def _loss_body(z_h, uv6_h, pi_h, pj_h, pk_h, ni_h, nj_h, nk_h, t6_h,
               out_reg, out_relu,
               ibuf, jbuf, kbuf, bi, bj, bk, u0, u1, u2, v0, v1, v2,
               tb0, tb1, tb2, accb):
    cid = lax.axis_index("c")
    sid = lax.axis_index("s")
    wid = cid * NS + sid
    iota = _iota16()
    z16 = jnp.zeros((LANES,), jnp.float32)
    tbs = (tb0, tb1, tb2)

    uvt = (u0, u1, u2, v0, v1, v2)
    for c in range(6):
        pltpu.sync_copy(uv6_h.at[c], uvt[c])

    acc_reg = z16
    acc_relu = z16
    fams = [
        (pi_h, pj_h, pk_h, (0, 4, 5), 1.0),
        (ni_h, nj_h, nk_h, (1, 2, 3), -1.0),
    ]
    for ih, jh, kh, blocks, sign in fams:
        def chunk(k, carry, ih=ih, jh=jh, kh=kh, blocks=blocks, sign=sign):
            acc_reg, acc_relu = carry
            base = wid * EPT + k * CH
            pltpu.sync_copy(ih.at[pl.ds(base, CH)], ibuf)
            pltpu.sync_copy(jh.at[pl.ds(base, CH)], jbuf)
            pltpu.sync_copy(kh.at[pl.ds(base, CH)], kbuf)
            pltpu.sync_copy(z_h.at[ibuf], bi)
            pltpu.sync_copy(z_h.at[jbuf], bj)
            pltpu.sync_copy(z_h.at[kbuf], bk)
            for o, blk in enumerate(blocks):
                pltpu.sync_copy(t6_h.at[blk].at[pl.ds(base, CH)], tbs[o])
            for g in range(CH // LANES):
                rows = iota + g * LANES

                def dstep(jj, c2):
                    aij, aik = c2
                    for dd in range(8):
                        col = jnp.zeros((LANES,), jnp.int32) + (jj * 8 + dd)
                        a = plsc.load_gather(bi, [rows, col])
                        b = plsc.load_gather(bj, [rows, col])
                        c = plsc.load_gather(bk, [rows, col])
                        aij = aij + (a - b) * (a - b)
                        aik = aik + (a - c) * (a - c)
                    return aij, aik

                aij, aik = lax.fori_loop(0, D // 8, dstep, (z16, z16))
                acc_relu = acc_relu + jnp.maximum(sign * (aij - aik), 0.0)

                validf = jnp.where(base + rows < E, 1.0, 0.0)
                sl = pl.ds(g * LANES, LANES)
                na, nb, nk_ = ibuf[sl], jbuf[sl], kbuf[sl]
                for o, (xa, xb) in enumerate(((na, nb), (na, nk_), (nb, nk_))):
                    p = [plsc.load_gather(uvt[ci], [xa])
                         + plsc.load_gather(uvt[3 + ci], [xb])
                         for ci in range(3)]
                    m = jnp.maximum(p[0], jnp.maximum(p[1], p[2]))
                    s = (jnp.exp(p[0] - m) + jnp.exp(p[1] - m)
                         + jnp.exp(p[2] - m))
                    ls = m + _fastlog(s)
                    t = tbs[o][pl.ds(g * LANES, LANES)]
                    pt = jnp.where(t == 0, p[0], jnp.where(t == 1, p[1], p[2]))
                    acc_reg = acc_reg + (ls - pt) * validf
            return acc_reg, acc_relu

        acc_reg, acc_relu = lax.fori_loop(0, NCHUNK, chunk,
                                          (acc_reg, acc_relu))

    accb[0, :] = acc_reg
    pltpu.sync_copy(accb, out_reg.at[pl.ds(wid, 1)])
    accb[0, :] = acc_relu
    pltpu.sync_copy(accb, out_relu.at[pl.ds(wid, 1)])


_loss_sc = pl.kernel(
    _loss_body,
    out_type=(jax.ShapeDtypeStruct((NTILES, 16), jnp.float32),
              jax.ShapeDtypeStruct((NTILES, 16), jnp.float32)),
    mesh=_mesh,
    compiler_params=pltpu.CompilerParams(needs_layout_passes=False,
                                         disable_bounds_checks=True),
    scratch_types=[
        pltpu.VMEM((CH,), jnp.int32),
        pltpu.VMEM((CH,), jnp.int32),
        pltpu.VMEM((CH,), jnp.int32),
        pltpu.VMEM((CH, D), jnp.float32),
        pltpu.VMEM((CH, D), jnp.float32),
        pltpu.VMEM((CH, D), jnp.float32),
        pltpu.VMEM((NPAD,), jnp.float32),
        pltpu.VMEM((NPAD,), jnp.float32),
        pltpu.VMEM((NPAD,), jnp.float32),
        pltpu.VMEM((NPAD,), jnp.float32),
        pltpu.VMEM((NPAD,), jnp.float32),
        pltpu.VMEM((NPAD,), jnp.float32),
        pltpu.VMEM((CH,), jnp.int32),
        pltpu.VMEM((CH,), jnp.int32),
        pltpu.VMEM((CH,), jnp.int32),
        pltpu.VMEM((1, 16), jnp.float32),
    ],
)


def kernel(X, positive_edges, negative_edges, target, pos_surrogates,
           neg_surrogates, W_pos_base, b_pos_base, W_neg_base, b_neg_base,
           W_pos_deep, b_pos_deep, W_neg_deep, b_neg_deep,
           regression_weights):
    f32 = jnp.float32
    padE = lambda a: jnp.pad(a, (0, EPAD - E))
    rp, cp_ = padE(positive_edges[0]), padE(positive_edges[1])
    rn, cn_ = padE(negative_edges[0]), padE(negative_edges[1])
    pk = padE(pos_surrogates)
    nk = padE(neg_surrogates)
    t6 = jnp.pad(target.reshape(6, E), ((0, 0), (0, EPAD - E)))
    Xp = jnp.pad(X, ((0, NPAD - N), (0, 0)))

    zrow = jnp.zeros((RPT, D), f32)
    zcnt = jnp.zeros((NPAD,), f32)

    sp, cp32 = _agg_cnt(Xp, rp, cp_, zrow, zcnt)
    sn, cn32 = _agg_cnt(Xp, rn, cn_, zrow, zcnt)
    cpc = jnp.sum(cp32, axis=0)[:, None]
    cnc = jnp.sum(cn32, axis=0)[:, None]
    H = _base_tc(sp, cpc, sn, cnc, Xp,
                 W_pos_base, b_pos_base.reshape(1, L1),
                 W_neg_base, b_neg_base.reshape(1, L1))
    (spd,) = _agg_sum(H, rp, cp_, zrow)
    (snd,) = _agg_sum(H, rn, cn_, zrow)
    rw8 = jnp.pad(jnp.concatenate(
        [regression_weights[:D], regression_weights[D:]], axis=1),
        ((0, 0), (0, 2)))
    Z, UVo = _deep_tc(spd, snd, cpc, cnc, H,
                      W_pos_deep, b_pos_deep.reshape(1, L2),
                      W_neg_deep, b_neg_deep.reshape(1, L2), rw8)
    uv6 = UVo[:, :6].T
    reg, relu = _loss_sc(Z, uv6, rp, cp_, pk, rn, cn_, nk, t6)
    z = Z[:N]
    loss = jnp.sum(reg) / (6.0 * E) + jnp.sum(relu) / E
    return loss, z
